# baseline (speedup 1.0000x reference)
import functools

import jax
import jax.numpy as jnp
from jax import lax
from jax.experimental import pallas as pl
from jax.experimental.pallas import tpu as pltpu
from jax.experimental.pallas import tpu_sc as plsc

L = 16


def _tc_body(z_ref, w1a_ref, w1b_ref, w1c_ref, xn_ref, b1_ref, eps_ref, b2_ref,
             p_ref, q_ref, lg_ref):
    beff = (
        jnp.dot(xn_ref[...], w1c_ref[...], preferred_element_type=jnp.float32)
        + b1_ref[...]
    )
    zb = z_ref[...]
    p_ref[...] = jnp.dot(zb, w1a_ref[...], preferred_element_type=jnp.float32) + beff
    q_ref[...] = jnp.dot(zb, w1b_ref[...], preferred_element_type=jnp.float32)

    @pl.when(pl.program_id(0) == 0)
    def _():
        bias = 0.0001
        es = eps_ref[...] * (bias - (1.0 - bias)) + (1.0 - bias)
        lg_ref[...] = (jnp.log(es) - jnp.log1p(-es) + b2_ref[0, 0]) * 0.2


def _tc_stage(z, w1a, w1b, w1c, xnode, b1r, eps2d, b2r, grid):
    n, c = z.shape
    h = w1a.shape[1]
    er, ec = eps2d.shape
    bn = n // grid
    br = er // grid
    return pl.pallas_call(
        _tc_body,
        grid=(grid,),
        in_specs=[
            pl.BlockSpec((bn, c), lambda i: (i, 0)),
            pl.BlockSpec((c, h), lambda i: (0, 0)),
            pl.BlockSpec((c, h), lambda i: (0, 0)),
            pl.BlockSpec((c, h), lambda i: (0, 0)),
            pl.BlockSpec((1, c), lambda i: (0, 0)),
            pl.BlockSpec((1, h), lambda i: (0, 0)),
            pl.BlockSpec((er, ec), lambda i: (0, 0)),
            pl.BlockSpec((1, 1), lambda i: (0, 0)),
        ],
        out_specs=[
            pl.BlockSpec((bn, h), lambda i: (i, 0)),
            pl.BlockSpec((bn, h), lambda i: (i, 0)),
            pl.BlockSpec((er, ec), lambda i: (0, 0)),
        ],
        out_shape=[
            jax.ShapeDtypeStruct((n, h), jnp.float32),
            jax.ShapeDtypeStruct((n, h), jnp.float32),
            jax.ShapeDtypeStruct((er, ec), jnp.float32),
        ],
    )(z, w1a, w1b, w1c, xnode, b1r, eps2d, b2r)


def _make_sc_kernel(e, n, h, nw, b):
    mesh = plsc.VectorSubcoreMesh(core_axis_name="c", subcore_axis_name="s")
    span = e // nw
    nc = span // b
    tail = span - nc * b
    nbuf = 2

    @functools.partial(
        pl.kernel,
        out_type=jax.ShapeDtypeStruct((e,), jnp.float32),
        mesh=mesh,
        compiler_params=pltpu.CompilerParams(
            use_tc_tiling_on_sc=False, needs_layout_passes=False
        ),
        scratch_types=[
            pltpu.VMEM((span,), jnp.int32),
            pltpu.VMEM((span,), jnp.int32),
            pltpu.VMEM((span,), jnp.float32),
            [pltpu.VMEM((b, h), jnp.float32) for _ in range(nbuf)],
            [pltpu.VMEM((b, h), jnp.float32) for _ in range(nbuf)],
            [pltpu.VMEM((b,), jnp.float32) for _ in range(nbuf)],
            pltpu.VMEM((L, L + 1), jnp.float32),
            pltpu.VMEM((h,), jnp.float32),
            pltpu.VMEM_SHARED((n, h), jnp.float32),
            pltpu.VMEM_SHARED((n, h), jnp.float32),
            [pltpu.SemaphoreType.DMA for _ in range(nbuf)],
            [pltpu.SemaphoreType.DMA for _ in range(nbuf)],
            [pltpu.SemaphoreType.DMA for _ in range(nbuf)],
        ],
    )
    def sc_edge_mlp(
        p_hbm, q_hbm, rows_hbm, cols_hbm, lg_hbm, w2_hbm, out_hbm,
        rows_v, cols_v, out_v, bufs_p, bufs_q, lgbs, tbuf, w2l, sh_p, sh_q,
        sems_p, sems_q, sems_lg,
    ):
        n_cores = lax.axis_size("c")
        n_sub = lax.axis_size("s")
        sid = lax.axis_index("s")
        wid = sid * n_cores + lax.axis_index("c")
        base = wid * span
        rps = n // n_sub
        pltpu.sync_copy(p_hbm.at[pl.ds(sid * rps, rps)], sh_p.at[pl.ds(sid * rps, rps)])
        pltpu.sync_copy(q_hbm.at[pl.ds(sid * rps, rps)], sh_q.at[pl.ds(sid * rps, rps)])
        pltpu.sync_copy(rows_hbm.at[pl.ds(base, span)], rows_v)
        pltpu.sync_copy(cols_hbm.at[pl.ds(base, span)], cols_v)
        pltpu.sync_copy(w2_hbm, w2l)
        w2vs = [w2l[pl.ds(j * L, L)] for j in range(h // L)]
        plsc.subcore_barrier()

        def copies(ci, slot, nrow):
            return (
                pltpu.make_async_copy(
                    sh_p.at[rows_v.at[pl.ds(ci * b, nrow)]],
                    bufs_p[slot].at[pl.ds(0, nrow)], sems_p[slot],
                ),
                pltpu.make_async_copy(
                    sh_q.at[cols_v.at[pl.ds(ci * b, nrow)]],
                    bufs_q[slot].at[pl.ds(0, nrow)], sems_q[slot],
                ),
                pltpu.make_async_copy(
                    lg_hbm.at[pl.ds(base + ci * b, nrow)],
                    lgbs[slot].at[pl.ds(0, nrow)], sems_lg[slot],
                ),
            )

        def start(ci, slot, nrow):
            for cp in copies(ci, slot, nrow):
                cp.start()

        def wait(ci, slot, nrow):
            for cp in copies(ci, slot, nrow):
                cp.wait()

        def _tree_sum(vals):
            while len(vals) > 1:
                vals = [
                    vals[i] + vals[i + 1] if i + 1 < len(vals) else vals[i]
                    for i in range(0, len(vals), 2)
                ]
            return vals[0]

        ii = lax.iota(jnp.int32, L)

        def one_group(ci, slot, g):
            buf_p, buf_q = bufs_p[slot], bufs_q[slot]
            for ei in range(L):
                row = g * L + ei
                parts = []
                for j in range(h // L):
                    p = buf_p[row, pl.ds(j * L, L)]
                    q = buf_q[row, pl.ds(j * L, L)]
                    parts.append(jnp.maximum(p + q, 0.0) * w2vs[j])
                tbuf[ei, pl.ds(0, L)] = _tree_sum(parts)
            cvs = [
                plsc.load_gather(tbuf, [ii, jnp.full((L,), cc, jnp.int32)])
                for cc in range(L)
            ]
            off = ci * b + g * L
            out_v[pl.ds(off, L)] = lgbs[slot][pl.ds(g * L, L)] + _tree_sum(cvs)

        def compute(ci, slot):
            def group_body(g, c2):
                one_group(ci, slot, g)
                return c2

            lax.fori_loop(0, b // L, group_body, 0)

        start(0, 0, b)

        def body(i2, carry):
            for s in range(nbuf):
                ci = i2 * nbuf + s

                @pl.when(ci + 1 < nc)
                def _():
                    start(ci + 1, (s + 1) % nbuf, b)

                wait(ci, s, b)
                compute(ci, s)
            return carry

        lax.fori_loop(0, nc // nbuf, body, 0)
        if tail:
            start(nc, 0, tail)
            wait(nc, 0, tail)
            for g in range(tail // L):
                one_group(nc, 0, g)
        pltpu.sync_copy(out_v, out_hbm.at[pl.ds(base, span)])

    return sc_edge_mlp


def kernel(z, edge_index, node_id, eps, W1, b1, W2, b2):
    n, c = z.shape
    e = edge_index.shape[1]
    h = W1.shape[1]

    info = plsc.get_sparse_core_info()
    nw = info.num_cores * info.num_subcores
    b = 64

    w1a = W1[:c]
    w1b = W1[c : 2 * c]
    w1c = W1[2 * c :]
    xnode = lax.dynamic_slice_in_dim(z, node_id, 1, axis=0)
    b1r = b1.reshape(1, h)
    eps2d = eps.reshape(e // 128, 128)

    p_tab, q_tab, lg2d = _tc_stage(
        z, w1a, w1b, w1c, xnode, b1r, eps2d, b2.reshape(1, 1), 10
    )
    w2v = W2[:, 0] * 0.2

    sc_fn = _make_sc_kernel(e, n, h, nw, b)
    return sc_fn(p_tab, q_tab, edge_index[0], edge_index[1], lg2d.reshape(e), w2v)
